# baseline (speedup 1.0000x reference)
import functools

import jax
import jax.numpy as jnp
from jax import lax
from jax.experimental import pallas as pl
from jax.experimental.pallas import tpu as pltpu
from jax.experimental.pallas import tpu_sc as plsc

N = 10000
E = 160000
H = 256
NB = 4
SUB = H // NB
NREL = 100

NC = 2
NS = 16
NW = NC * NS
CH = 128

TE = 256
E_PAD = 188416
NT = E_PAD // TE
NR = 10240
DUMMY = N
COLH = H // NC
RPT = NR // NS
DEGW = 128
TN = 1024


def _mesh():
  return plsc.VectorSubcoreMesh(core_axis_name="c", subcore_axis_name="s",
                                num_cores=NC, num_subcores=NS)


def _sc_gather(out_rows):
  n_chunks = out_rows // (NS * CH)
  assert n_chunks * NS * CH == out_rows

  @functools.partial(
      pl.kernel,
      out_type=jax.ShapeDtypeStruct((NC, out_rows, COLH), jnp.float32),
      mesh=_mesh(),
      scratch_types=[
          pltpu.VMEM((2, CH), jnp.int32),
          pltpu.VMEM((2, CH, COLH), jnp.float32),
          pltpu.VMEM_SHARED((NR, COLH), jnp.float32),
          pltpu.SemaphoreType.DMA,
          pltpu.SemaphoreType.DMA,
          pltpu.SemaphoreType.DMA,
      ],
  )
  def k(table_hbm, idx_hbm, out_hbm, idx2, buf2, table_sh, sem_i, sem_g,
        sem_o):
    c = lax.axis_index("c")
    s = lax.axis_index("s")
    rbase = s * RPT

    def stage(i, carry):
      r0 = rbase + i * CH
      pltpu.sync_copy(table_hbm.at[c, pl.ds(r0, CH)], buf2.at[0])
      pltpu.sync_copy(buf2.at[0], table_sh.at[pl.ds(r0, CH)])
      return carry

    lax.fori_loop(0, RPT // CH, stage, 0)
    plsc.subcore_barrier()

    base = s * n_chunks
    pltpu.sync_copy(idx_hbm.at[pl.ds(base * CH, CH)], idx2.at[0])

    def body(i, carry):
      p = lax.rem(i, 2)
      e0 = (base + i) * CH

      @pl.when(i >= 2)
      def _():
        pltpu.make_async_copy(
            buf2.at[p], out_hbm.at[c, pl.ds(e0, CH)], sem_o
        ).wait()

      @pl.when(i + 1 < n_chunks)
      def _():
        pltpu.async_copy(
            idx_hbm.at[pl.ds(e0 + CH, CH)], idx2.at[1 - p], sem_i
        )

      pltpu.async_copy(table_sh.at[idx2.at[p]], buf2.at[p], sem_g).wait()
      pltpu.async_copy(buf2.at[p], out_hbm.at[c, pl.ds(e0, CH)], sem_o)

      @pl.when(i + 1 < n_chunks)
      def _():
        pltpu.make_async_copy(
            idx_hbm.at[pl.ds(e0 + CH, CH)], idx2.at[1 - p], sem_i
        ).wait()

      return carry

    lax.fori_loop(0, n_chunks, body, 0)

    def drain(i, carry):
      pltpu.make_async_copy(
          buf2.at[0], out_hbm.at[c, pl.ds(base * CH, CH)], sem_o
      ).wait()
      return carry

    lax.fori_loop(0, min(n_chunks, 2), drain, 0)

  return k


def _sc_scatter():
  n_chunks = E_PAD // (NS * CH)
  zc = RPT // CH

  @functools.partial(
      pl.kernel,
      out_type=jax.ShapeDtypeStruct((NC, NR, COLH), jnp.float32),
      mesh=_mesh(),
      scratch_types=[
          pltpu.VMEM((2, CH), jnp.int32),
          pltpu.VMEM((2, CH, COLH), jnp.float32),
          pltpu.VMEM_SHARED((NR, COLH), jnp.float32),
          pltpu.SemaphoreType.DMA,
          pltpu.SemaphoreType.DMA,
      ],
  )
  def k(msg_hbm, dst_hbm, zeros_hbm, out_hbm, idx2, buf2, table_sh, sem_i,
        sem_m):
    c = lax.axis_index("c")
    s = lax.axis_index("s")
    rbase = s * RPT

    pltpu.sync_copy(zeros_hbm, buf2.at[0])

    def zbody(i, carry):
      r0 = rbase + i * CH
      pltpu.sync_copy(buf2.at[0], table_sh.at[pl.ds(r0, CH)])
      return carry

    lax.fori_loop(0, zc, zbody, 0)
    plsc.subcore_barrier()

    base = s * n_chunks
    pltpu.sync_copy(dst_hbm.at[pl.ds(base * CH, CH)], idx2.at[0])
    pltpu.sync_copy(msg_hbm.at[c, pl.ds(base * CH, CH)], buf2.at[0])

    def body(i, carry):
      p = lax.rem(i, 2)
      e0 = (base + i) * CH

      @pl.when(i + 1 < n_chunks)
      def _():
        pltpu.async_copy(
            dst_hbm.at[pl.ds(e0 + CH, CH)], idx2.at[1 - p], sem_i
        )
        pltpu.async_copy(
            msg_hbm.at[c, pl.ds(e0 + CH, CH)], buf2.at[1 - p], sem_m
        )

      pltpu.sync_copy(buf2.at[p], table_sh.at[idx2.at[p]], add=True)

      @pl.when(i + 1 < n_chunks)
      def _():
        pltpu.make_async_copy(
            dst_hbm.at[pl.ds(e0 + CH, CH)], idx2.at[1 - p], sem_i
        ).wait()
        pltpu.make_async_copy(
            msg_hbm.at[c, pl.ds(e0 + CH, CH)], buf2.at[1 - p], sem_m
        ).wait()

      return carry

    lax.fori_loop(0, n_chunks, body, 0)
    plsc.subcore_barrier()

    def obody(i, carry):
      r0 = rbase + i * CH
      pltpu.sync_copy(table_sh.at[pl.ds(r0, CH)], buf2.at[0])
      pltpu.sync_copy(buf2.at[0], out_hbm.at[c, pl.ds(r0, CH)])
      return carry

    lax.fori_loop(0, zc, obody, 0)

  return k


E_DEG = 163840


def _sc_degree():
  half = E_DEG // NC
  n_chunks = half // (NS * CH)
  zc = RPT // CH

  @functools.partial(
      pl.kernel,
      out_type=jax.ShapeDtypeStruct((NC, NR, DEGW), jnp.float32),
      mesh=_mesh(),
      scratch_types=[
          pltpu.VMEM((CH,), jnp.int32),
          pltpu.VMEM((CH, DEGW), jnp.float32),
          pltpu.VMEM((CH, DEGW), jnp.float32),
          pltpu.VMEM_SHARED((NR, DEGW), jnp.float32),
          pltpu.SemaphoreType.DMA,
      ],
  )
  def k(dst_hbm, zeros_hbm, ones_hbm, out_hbm, idx_v, buf_v, ones_v, table_sh,
        sem):
    c = lax.axis_index("c")
    s = lax.axis_index("s")
    rbase = s * RPT
    pltpu.sync_copy(ones_hbm, ones_v)
    pltpu.sync_copy(zeros_hbm, buf_v)

    def zbody(i, carry):
      r0 = rbase + i * CH
      pltpu.sync_copy(buf_v, table_sh.at[pl.ds(r0, CH)])
      return carry

    lax.fori_loop(0, zc, zbody, 0)
    plsc.subcore_barrier()

    def body(i, carry):
      e0 = c * half + (s * n_chunks + i) * CH
      pltpu.sync_copy(dst_hbm.at[pl.ds(e0, CH)], idx_v)
      pltpu.sync_copy(ones_v, table_sh.at[idx_v], add=True)
      return carry

    lax.fori_loop(0, n_chunks, body, 0)
    plsc.subcore_barrier()

    def obody(i, carry):
      r0 = rbase + i * CH
      pltpu.sync_copy(table_sh.at[pl.ds(r0, CH)], buf_v)
      pltpu.sync_copy(buf_v, out_hbm.at[c, pl.ds(r0, CH)])
      return carry

    lax.fori_loop(0, zc, obody, 0)

  return k


def _tc_matmul():
  def body(rel_ref, hs_ref, w_ref, o_ref):
    x = jnp.concatenate([hs_ref[0], hs_ref[1]], axis=1).astype(jnp.bfloat16)
    y = jnp.dot(x, w_ref[0], preferred_element_type=jnp.float32)
    o_ref[0] = y[:, :COLH]
    o_ref[1] = y[:, COLH:]

  grid_spec = pltpu.PrefetchScalarGridSpec(
      num_scalar_prefetch=1,
      grid=(NT,),
      in_specs=[
          pl.BlockSpec((NC, TE, COLH), lambda i, rel: (0, i, 0)),
          pl.BlockSpec((1, H, H), lambda i, rel: (rel[i], 0, 0)),
      ],
      out_specs=pl.BlockSpec((NC, TE, COLH), lambda i, rel: (0, i, 0)),
  )
  return pl.pallas_call(
      body,
      grid_spec=grid_spec,
      out_shape=jax.ShapeDtypeStruct((NC, E_PAD, COLH), jnp.float32),
  )


def _tc_combine(act):
  def body(agg_ref, deg_ref, h_ref, w_ref, o_ref):
    deg = deg_ref[0, :, 0:1] + deg_ref[1, :, 0:1]
    norm = 1.0 / jnp.maximum(deg, 1.0)
    agg = jnp.concatenate([agg_ref[0], agg_ref[1]], axis=1)
    h = jnp.concatenate([h_ref[0], h_ref[1]], axis=1).astype(jnp.bfloat16)
    out = agg * norm + jnp.dot(h, w_ref[...], preferred_element_type=jnp.float32)
    if act:
      out = jnp.maximum(out, 0.0)
    o_ref[0] = out[:, :COLH]
    o_ref[1] = out[:, COLH:]

  return pl.pallas_call(
      body,
      grid=(NR // TN,),
      in_specs=[
          pl.BlockSpec((NC, TN, COLH), lambda i: (0, i, 0)),
          pl.BlockSpec((NC, TN, DEGW), lambda i: (0, i, 0)),
          pl.BlockSpec((NC, TN, COLH), lambda i: (0, i, 0)),
          pl.BlockSpec((H, H), lambda i: (0, 0)),
      ],
      out_specs=pl.BlockSpec((NC, TN, COLH), lambda i: (0, i, 0)),
      out_shape=jax.ShapeDtypeStruct((NC, NR, COLH), jnp.float32),
  )


def _blockdiag(W):
  Wf = jnp.zeros((NREL, NB, SUB, NB, SUB), W.dtype)
  for b in range(NB):
    Wf = Wf.at[:, b, :, b, :].set(W[:, b])
  return Wf.reshape(NREL, H, H)


def kernel(node_embeds, W1, loop1, W2, loop2, edge_index, edge_type, init_ids):
  et = edge_type.astype(jnp.int32)
  src = edge_index[0].astype(jnp.int32)
  dst = edge_index[1].astype(jnp.int32)
  iid = init_ids.astype(jnp.int32)

  counts = jnp.sum(
      (et[:, None] == jnp.arange(NREL, dtype=jnp.int32)[None, :]).astype(
          jnp.float32
      ),
      axis=0,
  ).astype(jnp.int32)
  pcounts = ((counts + TE - 1) // TE) * TE
  gaps = pcounts - counts
  gapend = jnp.cumsum(gaps)
  padn = E_PAD - E
  pad_rel = jnp.searchsorted(
      gapend, jnp.arange(padn, dtype=jnp.int32), side="right"
  ).astype(jnp.int32)
  keys = jnp.concatenate([et, pad_rel])
  packed = jnp.concatenate(
      [src * 16384 + dst, jnp.full((padn,), DUMMY, jnp.int32)]
  )
  _, packed_s = lax.sort((keys, packed), num_keys=1)
  src_pad = packed_s // 16384
  dst_pad = packed_s % 16384
  pstarts = jnp.concatenate(
      [jnp.zeros((1,), jnp.int32), jnp.cumsum(pcounts)[:-1]]
  )
  tile_rel = jnp.clip(
      jnp.searchsorted(
          pstarts, jnp.arange(NT, dtype=jnp.int32) * TE, side="right"
      ).astype(jnp.int32) - 1,
      0,
      NREL - 1,
  )

  iid_pad = jnp.concatenate([iid, jnp.zeros((NR - N,), jnp.int32)])
  zeros_col = jnp.zeros((CH, COLH), jnp.float32)
  ones_deg = jnp.ones((CH, DEGW), jnp.float32)
  Wd1 = _blockdiag(W1).astype(jnp.bfloat16)
  Wd2 = _blockdiag(W2).astype(jnp.bfloat16)
  loop1 = loop1.astype(jnp.bfloat16)
  loop2 = loop2.astype(jnp.bfloat16)

  ne_pad = jnp.concatenate(
      [node_embeds, jnp.zeros((NR - N, H), jnp.float32)]
  )
  ne_plane = jnp.stack([ne_pad[:, :COLH], ne_pad[:, COLH:]])

  gather_init = _sc_gather(NR)
  gather_edges = _sc_gather(E_PAD)
  scatter = _sc_scatter()
  degree = _sc_degree()
  matmul = _tc_matmul()

  dst_deg = jnp.concatenate(
      [dst, jnp.full((E_DEG - E,), DUMMY, jnp.int32)]
  )
  h0 = gather_init(ne_plane, iid_pad)
  deg = degree(dst_deg, zeros_col, ones_deg)

  hs1 = gather_edges(h0, src_pad)
  msg1 = matmul(tile_rel, hs1, Wd1)
  agg1 = scatter(msg1, dst_pad, zeros_col)
  h1 = _tc_combine(True)(agg1, deg, h0, loop1)

  hs2 = gather_edges(h1, src_pad)
  msg2 = matmul(tile_rel, hs2, Wd2)
  agg2 = scatter(msg2, dst_pad, zeros_col)
  out = _tc_combine(False)(agg2, deg, h1, loop2)
  return jnp.concatenate([out[0], out[1]], axis=1)[:N]


# speedup vs baseline: 1.0084x; 1.0084x over previous
import functools

import jax
import jax.numpy as jnp
from jax import lax
from jax.experimental import pallas as pl
from jax.experimental.pallas import tpu as pltpu
from jax.experimental.pallas import tpu_sc as plsc

N = 10000
E = 160000
H = 256
NB = 4
SUB = H // NB
NREL = 100

NC = 2
NS = 16
NW = NC * NS
CH = 128

TE = 256
E_PAD = 188416
NT = E_PAD // TE
NR = 10240
DUMMY = N
COLH = H // NC
RPT = NR // NS
DEGW = 128
TN = 1024


def _mesh():
  return plsc.VectorSubcoreMesh(core_axis_name="c", subcore_axis_name="s",
                                num_cores=NC, num_subcores=NS)


def _sc_gather(out_rows):
  n_chunks = out_rows // (NS * CH)
  assert n_chunks * NS * CH == out_rows

  @functools.partial(
      pl.kernel,
      out_type=jax.ShapeDtypeStruct((NC, out_rows, COLH), jnp.float32),
      mesh=_mesh(),
      scratch_types=[
          pltpu.VMEM((2, CH), jnp.int32),
          pltpu.VMEM((2, CH, COLH), jnp.float32),
          pltpu.VMEM_SHARED((NR, COLH), jnp.float32),
          pltpu.SemaphoreType.DMA,
          pltpu.SemaphoreType.DMA,
          pltpu.SemaphoreType.DMA,
      ],
  )
  def k(table_hbm, idx_hbm, out_hbm, idx2, buf2, table_sh, sem_i, sem_g,
        sem_o):
    c = lax.axis_index("c")
    s = lax.axis_index("s")
    rbase = s * RPT

    def stage(i, carry):
      r0 = rbase + i * CH
      pltpu.sync_copy(table_hbm.at[c, pl.ds(r0, CH)], buf2.at[0])
      pltpu.sync_copy(buf2.at[0], table_sh.at[pl.ds(r0, CH)])
      return carry

    lax.fori_loop(0, RPT // CH, stage, 0)
    plsc.subcore_barrier()

    base = s * n_chunks
    pltpu.sync_copy(idx_hbm.at[pl.ds(base * CH, CH)], idx2.at[0])

    def body(i, carry):
      p = lax.rem(i, 2)
      e0 = (base + i) * CH

      @pl.when(i >= 2)
      def _():
        pltpu.make_async_copy(
            buf2.at[p], out_hbm.at[c, pl.ds(e0, CH)], sem_o
        ).wait()

      @pl.when(i + 1 < n_chunks)
      def _():
        pltpu.async_copy(
            idx_hbm.at[pl.ds(e0 + CH, CH)], idx2.at[1 - p], sem_i
        )

      pltpu.async_copy(table_sh.at[idx2.at[p]], buf2.at[p], sem_g).wait()
      pltpu.async_copy(buf2.at[p], out_hbm.at[c, pl.ds(e0, CH)], sem_o)

      @pl.when(i + 1 < n_chunks)
      def _():
        pltpu.make_async_copy(
            idx_hbm.at[pl.ds(e0 + CH, CH)], idx2.at[1 - p], sem_i
        ).wait()

      return carry

    lax.fori_loop(0, n_chunks, body, 0)

    def drain(i, carry):
      pltpu.make_async_copy(
          buf2.at[0], out_hbm.at[c, pl.ds(base * CH, CH)], sem_o
      ).wait()
      return carry

    lax.fori_loop(0, min(n_chunks, 2), drain, 0)

  return k


def _sc_scatter():
  n_chunks = E_PAD // (NS * CH)
  zc = RPT // CH

  @functools.partial(
      pl.kernel,
      out_type=jax.ShapeDtypeStruct((NC, NR, COLH), jnp.float32),
      mesh=_mesh(),
      scratch_types=[
          pltpu.VMEM((2, CH), jnp.int32),
          pltpu.VMEM((2, CH, COLH), jnp.float32),
          pltpu.VMEM_SHARED((NR, COLH), jnp.float32),
          pltpu.SemaphoreType.DMA,
          pltpu.SemaphoreType.DMA,
      ],
  )
  def k(msg_hbm, dst_hbm, zeros_hbm, out_hbm, idx2, buf2, table_sh, sem_i,
        sem_m):
    c = lax.axis_index("c")
    s = lax.axis_index("s")
    rbase = s * RPT

    pltpu.sync_copy(zeros_hbm, buf2.at[0])

    def zbody(i, carry):
      r0 = rbase + i * CH
      pltpu.sync_copy(buf2.at[0], table_sh.at[pl.ds(r0, CH)])
      return carry

    lax.fori_loop(0, zc, zbody, 0)
    plsc.subcore_barrier()

    base = s * n_chunks
    pltpu.sync_copy(dst_hbm.at[pl.ds(base * CH, CH)], idx2.at[0])
    pltpu.sync_copy(msg_hbm.at[c, pl.ds(base * CH, CH)], buf2.at[0])

    def body(i, carry):
      p = lax.rem(i, 2)
      e0 = (base + i) * CH

      @pl.when(i + 1 < n_chunks)
      def _():
        pltpu.async_copy(
            dst_hbm.at[pl.ds(e0 + CH, CH)], idx2.at[1 - p], sem_i
        )
        pltpu.async_copy(
            msg_hbm.at[c, pl.ds(e0 + CH, CH)], buf2.at[1 - p], sem_m
        )

      pltpu.sync_copy(buf2.at[p], table_sh.at[idx2.at[p]], add=True)

      @pl.when(i + 1 < n_chunks)
      def _():
        pltpu.make_async_copy(
            dst_hbm.at[pl.ds(e0 + CH, CH)], idx2.at[1 - p], sem_i
        ).wait()
        pltpu.make_async_copy(
            msg_hbm.at[c, pl.ds(e0 + CH, CH)], buf2.at[1 - p], sem_m
        ).wait()

      return carry

    lax.fori_loop(0, n_chunks, body, 0)
    plsc.subcore_barrier()

    def obody(i, carry):
      r0 = rbase + i * CH
      pltpu.sync_copy(table_sh.at[pl.ds(r0, CH)], buf2.at[0])
      pltpu.sync_copy(buf2.at[0], out_hbm.at[c, pl.ds(r0, CH)])
      return carry

    lax.fori_loop(0, zc, obody, 0)

  return k


def _sc_degree():
  half = E_PAD // NC
  n_chunks = half // (NS * CH)
  zc = RPT // CH

  @functools.partial(
      pl.kernel,
      out_type=jax.ShapeDtypeStruct((NC, NR, DEGW), jnp.float32),
      mesh=_mesh(),
      scratch_types=[
          pltpu.VMEM((CH,), jnp.int32),
          pltpu.VMEM((CH, DEGW), jnp.float32),
          pltpu.VMEM((CH, DEGW), jnp.float32),
          pltpu.VMEM_SHARED((NR, DEGW), jnp.float32),
          pltpu.SemaphoreType.DMA,
      ],
  )
  def k(dst_hbm, zeros_hbm, ones_hbm, out_hbm, idx_v, buf_v, ones_v, table_sh,
        sem):
    c = lax.axis_index("c")
    s = lax.axis_index("s")
    rbase = s * RPT
    pltpu.sync_copy(ones_hbm, ones_v)
    pltpu.sync_copy(zeros_hbm, buf_v)

    def zbody(i, carry):
      r0 = rbase + i * CH
      pltpu.sync_copy(buf_v, table_sh.at[pl.ds(r0, CH)])
      return carry

    lax.fori_loop(0, zc, zbody, 0)
    plsc.subcore_barrier()

    def body(i, carry):
      e0 = c * half + (s * n_chunks + i) * CH
      pltpu.sync_copy(dst_hbm.at[pl.ds(e0, CH)], idx_v)
      pltpu.sync_copy(ones_v, table_sh.at[idx_v], add=True)
      return carry

    lax.fori_loop(0, n_chunks, body, 0)
    plsc.subcore_barrier()

    def obody(i, carry):
      r0 = rbase + i * CH
      pltpu.sync_copy(table_sh.at[pl.ds(r0, CH)], buf_v)
      pltpu.sync_copy(buf_v, out_hbm.at[c, pl.ds(r0, CH)])
      return carry

    lax.fori_loop(0, zc, obody, 0)

  return k


def _tc_matmul():
  def body(rel_ref, hs_ref, w_ref, o_ref):
    x = jnp.concatenate([hs_ref[0], hs_ref[1]], axis=1).astype(jnp.bfloat16)
    y = jnp.dot(x, w_ref[0], preferred_element_type=jnp.float32)
    o_ref[0] = y[:, :COLH]
    o_ref[1] = y[:, COLH:]

  grid_spec = pltpu.PrefetchScalarGridSpec(
      num_scalar_prefetch=1,
      grid=(NT,),
      in_specs=[
          pl.BlockSpec((NC, TE, COLH), lambda i, rel: (0, i, 0)),
          pl.BlockSpec((1, H, H), lambda i, rel: (rel[i], 0, 0)),
      ],
      out_specs=pl.BlockSpec((NC, TE, COLH), lambda i, rel: (0, i, 0)),
  )
  return pl.pallas_call(
      body,
      grid_spec=grid_spec,
      out_shape=jax.ShapeDtypeStruct((NC, E_PAD, COLH), jnp.float32),
  )


def _tc_combine(act):
  def body(agg_ref, deg_ref, h_ref, w_ref, o_ref):
    deg = deg_ref[0, :, 0:1] + deg_ref[1, :, 0:1]
    norm = 1.0 / jnp.maximum(deg, 1.0)
    agg = jnp.concatenate([agg_ref[0], agg_ref[1]], axis=1)
    h = jnp.concatenate([h_ref[0], h_ref[1]], axis=1).astype(jnp.bfloat16)
    out = agg * norm + jnp.dot(h, w_ref[...], preferred_element_type=jnp.float32)
    if act:
      out = jnp.maximum(out, 0.0)
    o_ref[0] = out[:, :COLH]
    o_ref[1] = out[:, COLH:]

  return pl.pallas_call(
      body,
      grid=(NR // TN,),
      in_specs=[
          pl.BlockSpec((NC, TN, COLH), lambda i: (0, i, 0)),
          pl.BlockSpec((NC, TN, DEGW), lambda i: (0, i, 0)),
          pl.BlockSpec((NC, TN, COLH), lambda i: (0, i, 0)),
          pl.BlockSpec((H, H), lambda i: (0, 0)),
      ],
      out_specs=pl.BlockSpec((NC, TN, COLH), lambda i: (0, i, 0)),
      out_shape=jax.ShapeDtypeStruct((NC, NR, COLH), jnp.float32),
  )


def _blockdiag(W):
  Wf = jnp.zeros((NREL, NB, SUB, NB, SUB), W.dtype)
  for b in range(NB):
    Wf = Wf.at[:, b, :, b, :].set(W[:, b])
  return Wf.reshape(NREL, H, H)


def kernel(node_embeds, W1, loop1, W2, loop2, edge_index, edge_type, init_ids):
  et = edge_type.astype(jnp.int32)
  src = edge_index[0].astype(jnp.int32)
  dst = edge_index[1].astype(jnp.int32)
  iid = init_ids.astype(jnp.int32)

  counts = jnp.sum(
      (et[:, None] == jnp.arange(NREL, dtype=jnp.int32)[None, :]).astype(
          jnp.float32
      ),
      axis=0,
  ).astype(jnp.int32)
  pcounts = ((counts + TE - 1) // TE) * TE
  gaps = pcounts - counts
  gapend = jnp.cumsum(gaps)
  padn = E_PAD - E
  pad_rel = jnp.searchsorted(
      gapend, jnp.arange(padn, dtype=jnp.int32), side="right"
  ).astype(jnp.int32)
  keys = jnp.concatenate([et, pad_rel])
  packed = jnp.concatenate(
      [src * 16384 + dst, jnp.full((padn,), DUMMY, jnp.int32)]
  )
  _, packed_s = lax.sort((keys, packed), num_keys=1)
  src_pad = packed_s // 16384
  dst_pad = packed_s % 16384
  pstarts = jnp.concatenate(
      [jnp.zeros((1,), jnp.int32), jnp.cumsum(pcounts)[:-1]]
  )
  tile_rel = jnp.clip(
      jnp.searchsorted(
          pstarts, jnp.arange(NT, dtype=jnp.int32) * TE, side="right"
      ).astype(jnp.int32) - 1,
      0,
      NREL - 1,
  )

  iid_pad = jnp.concatenate([iid, jnp.zeros((NR - N,), jnp.int32)])
  zeros_col = jnp.zeros((CH, COLH), jnp.float32)
  ones_deg = jnp.ones((CH, DEGW), jnp.float32)
  Wd1 = _blockdiag(W1).astype(jnp.bfloat16)
  Wd2 = _blockdiag(W2).astype(jnp.bfloat16)
  loop1 = loop1.astype(jnp.bfloat16)
  loop2 = loop2.astype(jnp.bfloat16)

  ne_pad = jnp.concatenate(
      [node_embeds, jnp.zeros((NR - N, H), jnp.float32)]
  )
  ne_plane = jnp.stack([ne_pad[:, :COLH], ne_pad[:, COLH:]])

  gather_init = _sc_gather(NR)
  gather_edges = _sc_gather(E_PAD)
  scatter = _sc_scatter()
  degree = _sc_degree()
  matmul = _tc_matmul()

  h0 = gather_init(ne_plane, iid_pad)
  deg = degree(dst_pad, zeros_col, ones_deg)

  hs1 = gather_edges(h0, src_pad)
  msg1 = matmul(tile_rel, hs1, Wd1)
  agg1 = scatter(msg1, dst_pad, zeros_col)
  h1 = _tc_combine(True)(agg1, deg, h0, loop1)

  hs2 = gather_edges(h1, src_pad)
  msg2 = matmul(tile_rel, hs2, Wd2)
  agg2 = scatter(msg2, dst_pad, zeros_col)
  out = _tc_combine(False)(agg2, deg, h1, loop2)
  return jnp.concatenate([out[0], out[1]], axis=1)[:N]
